# baseline (speedup 1.0000x reference)
import functools

import jax
import jax.numpy as jnp
from jax import lax
from jax.experimental import pallas as pl
from jax.experimental.pallas import tpu as pltpu
from jax.experimental.pallas import tpu_sc as plsc

B = 4
N = 307200
D = 16
NB_MATCH = 5000
NB_NONMATCH = 50000
NB_SAMPLE = NB_NONMATCH // NB_MATCH
ALPHA = 0.5
TOTAL = B * NB_NONMATCH
NW = 32
C = 320
CM = C // NB_SAMPLE
CPW = 20
ROWS_PER_W = C * CPW
PAD_TOTAL = NW * ROWS_PER_W
TBL = B * N
G = TBL // 8
MAXOFF = (B - 1) * N

_mesh = plsc.VectorSubcoreMesh(core_axis_name="c", subcore_axis_name="s")


@functools.partial(
    pl.kernel,
    mesh=_mesh,
    compiler_params=pltpu.CompilerParams(needs_layout_passes=False),
    out_type=jax.ShapeDtypeStruct((NW, 16), jnp.float32),
    scratch_types=[
        pltpu.VMEM((C,), jnp.int32),
        pltpu.VMEM((C,), jnp.int32),
        pltpu.VMEM((C,), jnp.int32),
        pltpu.VMEM((C,), jnp.int32),
        pltpu.VMEM((CM,), jnp.int32),
        pltpu.VMEM((CM,), jnp.int32),
        pltpu.VMEM((C, 128), jnp.float32),
        pltpu.VMEM((C, 128), jnp.float32),
        pltpu.VMEM((CM, 128), jnp.float32),
        pltpu.VMEM((16,), jnp.float32),
        pltpu.SemaphoreType.DMA,
    ],
)
def _triplet_sc(tblA, tblB, idxA_h, mB_h, idxBn_h, out_h,
                gA_v, wA_v, gBn_v, wBn_v, gBp_v, wBp_v,
                rA, rBn, rBp, accv, sem):
    wid = lax.axis_index("s") * 2 + lax.axis_index("c")

    def chunk_body(c, acc):
        s = pl.multiple_of(wid * ROWS_PER_W + c * C, C)
        sm = pl.multiple_of(s // NB_SAMPLE, CM)
        pltpu.sync_copy(idxA_h.at[pl.ds(s, C)], gA_v)
        pltpu.sync_copy(idxBn_h.at[pl.ds(s, C)], gBn_v)
        pltpu.sync_copy(mB_h.at[pl.ds(sm, CM)], gBp_v)

        def adj_body(j, carry):
            sl = pl.ds(j * 16, 16)
            iota = lax.iota(jnp.int32, 16)
            base = s + j * 16
            b0 = base // NB_NONMATCH
            r0 = base % NB_NONMATCH
            bump = jnp.where(iota >= NB_NONMATCH - r0, N, 0)
            off = jnp.minimum(b0 * N + bump, MAXOFF)
            va = gA_v[sl] + off
            wA_v[sl] = (va & 7) << 4
            gA_v[sl] = va >> 3
            vn = gBn_v[sl] + off
            wBn_v[sl] = (vn & 7) << 4
            gBn_v[sl] = vn >> 3
            return carry

        lax.fori_loop(0, C // 16, adj_body, 0)

        for j in range(CM // 16):
            sl = pl.ds(j * 16, 16)
            iota = lax.iota(jnp.int32, 16)
            basem = sm + j * 16
            b0 = basem // NB_MATCH
            r0 = basem % NB_MATCH
            bump = jnp.where(iota >= NB_MATCH - r0, N, 0)
            off = jnp.minimum(b0 * N + bump, MAXOFF)
            vp = gBp_v[sl] + off
            wBp_v[sl] = (vp & 7) << 4
            gBp_v[sl] = vp >> 3

        cpA = pltpu.async_copy(tblA.at[gA_v], rA, sem)
        cpBn = pltpu.async_copy(tblB.at[gBn_v], rBn, sem)
        cpBp = pltpu.async_copy(tblB.at[gBp_v], rBp, sem)
        cpA.wait()
        cpBn.wait()
        cpBp.wait()

        def row_body(i, a):
            iota = lax.iota(jnp.int32, 16)
            rvec = i * 16 + iota
            t0 = i * 16
            u0 = t0 // NB_SAMPLE
            ur = t0 % NB_SAMPLE
            uvec = (u0
                    + jnp.where(ur + iota >= NB_SAMPLE, 1, 0)
                    + jnp.where(ur + iota >= 2 * NB_SAMPLE, 1, 0))
            wA = wA_v[pl.ds(t0, 16)]
            wBn = wBn_v[pl.ds(t0, 16)]
            wBp = plsc.load_gather(wBp_v, [uvec])
            sacc = jnp.zeros((16,), jnp.float32)
            for d in range(D):
                aa = plsc.load_gather(rA, [rvec, wA + d])
                bn = plsc.load_gather(rBn, [rvec, wBn + d])
                bp = plsc.load_gather(rBp, [uvec, wBp + d])
                sacc = sacc + (bn - bp) * (aa + aa - bp - bn)
            return a + jnp.maximum(sacc + ALPHA, 0.0)

        return lax.fori_loop(0, C // 16, row_body, acc)

    acc = lax.fori_loop(0, CPW, chunk_body, jnp.zeros((16,), jnp.float32))
    accv[...] = acc
    pltpu.sync_copy(accv, out_h.at[wid])


def kernel(outA, outB, matchA, matchB, nonMatchA, nonMatchB, device):
    tblA = outA.reshape(G, 128)
    tblB = outB.reshape(G, 128)
    idxA = jnp.pad(nonMatchA.reshape(-1), (0, PAD_TOTAL - TOTAL))
    idxBn = jnp.pad(nonMatchB.reshape(-1), (0, PAD_TOTAL - TOTAL))
    mB = jnp.pad(matchB.reshape(-1), (0, (PAD_TOTAL - TOTAL) // NB_SAMPLE))
    partials = _triplet_sc(tblA, tblB, idxA, mB, idxBn)
    pad_fix = 0.5 * (PAD_TOTAL - TOTAL)
    return (jnp.sum(partials) - pad_fix) / NB_MATCH
